# Initial kernel scaffold; baseline (speedup 1.0000x reference)
import jax
import jax.numpy as jnp
from jax.experimental import pallas as pl


def kernel(feat_idx, label, emb_mean, emb_std, fc_w, fc_w_concat, mlp_p_W1, mlp_p_b1, mlp_p_W2, mlp_p_b2, mlp_q_W1, mlp_q_b1, mlp_q_W2, mlp_q_b2, log_alpha, rand_array):
    raise NotImplementedError("write your pallas kernel here")


# speedup vs baseline: 1.4352x; 1.4352x over previous
import jax
import jax.numpy as jnp
from jax.experimental import pallas as pl
from jax.experimental.pallas import tpu as pltpu

_B = 16384
_D = 64
_NCOL = 8
_NUM_EMB = 12
_PAIRS = [(i1, i2) for i1 in range(_NCOL) for i2 in range(i1 + 1, _NCOL)]
_BT = 1024


def _nasp_kernel(idx_ref, label_ref, mean_ref, std_ref, fcw_ref, fcc_ref,
                 mparams_ref, logalpha_ref, v_ref, inf_ref, rew_ref):
    f32 = jnp.float32
    idx = idx_ref[...]
    v = v_ref[...]
    mean_tab = mean_ref[...]
    sp_tab = jnp.log1p(jnp.exp(std_ref[...]))
    fcw = fcw_ref[...]
    fcc = fcc_ref[...]

    def gather12(tab, idx_col):
        out = jnp.broadcast_to(tab[0:1, :], idx_col.shape[:1] + (_D,))
        for r in range(1, _NUM_EMB):
            out = jnp.where(idx_col == r, tab[r:r + 1, :], out)
        return out

    def rb(x):
        u = jax.lax.bitcast_convert_type(x, jnp.uint32)
        r = (u + jnp.uint32(0x7FFF) + ((u >> 16) & jnp.uint32(1))) \
            & jnp.uint32(0xFFFF0000)
        return jax.lax.bitcast_convert_type(r, f32)

    def mlp(x, base):
        acc = None
        for h in range(8):
            z = mparams_ref[base, h] * x + mparams_ref[base + 1, h]
            t = jnp.tanh(z)
            term = mparams_ref[base + 2, h] * rb(t)
            acc = term if acc is None else acc + term
        return acc + mparams_ref[base + 3, 0]

    ep = [None] * _NCOL
    eq = [None] * _NCOL
    for c in range(_NCOL):
        idx_c = idx[:, c:c + 1]
        mu = gather12(mean_tab[c * 12:(c + 1) * 12, :], idx_c)
        s = gather12(sp_tab[c * 12:(c + 1) * 12, :], idx_c)
        e = mu + s * v * 0.01
        if c < _NCOL - 1:
            ep[c] = mlp(e, 0)
        if c > 0:
            eq[c] = mlp(e, 4)

    pos = jnp.int32(0)
    best = logalpha_ref[0, 0]
    for j in range(1, 5):
        lj = logalpha_ref[0, j]
        take = lj > best
        pos = jnp.where(take, jnp.int32(j), pos)
        best = jnp.where(take, lj, best)

    def elem_branch(k):
        def branch():
            acc0 = None
            acc1 = None
            for p, (i1, i2) in enumerate(_PAIRS):
                a, b = ep[i1], eq[i2]
                if k == 0:
                    comb = a + b
                elif k == 1:
                    comb = a * b
                elif k == 2:
                    comb = jnp.maximum(a, b)
                else:
                    comb = jnp.minimum(a, b)
                r = p * 8 + k * 2
                cb = rb(comb)
                t0 = cb * fcw[r:r + 1, :]
                t1 = cb * fcw[r + 1:r + 2, :]
                acc0 = t0 if acc0 is None else acc0 + t0
                acc1 = t1 if acc1 is None else acc1 + t1
            return acc0, acc1
        return branch

    def concat_branch():
        epb = [rb(e) if e is not None else None for e in ep]
        eqb = [rb(e) if e is not None else None for e in eq]
        acc0 = None
        acc1 = None
        for p, (i1, i2) in enumerate(_PAIRS):
            a, b = epb[i1], eqb[i2]
            r = p * 2
            t0 = a * fcc[r:r + 1, 0:64] + b * fcc[r:r + 1, 64:128]
            t1 = a * fcc[r + 1:r + 2, 0:64] + b * fcc[r + 1:r + 2, 64:128]
            acc0 = t0 if acc0 is None else acc0 + t0
            acc1 = t1 if acc1 is None else acc1 + t1
        return acc0, acc1

    acc0, acc1 = jax.lax.switch(
        pos, [elem_branch(0), elem_branch(1), elem_branch(2), elem_branch(3),
              concat_branch])
    s0 = jnp.sum(acc0, axis=1, keepdims=True)
    s1 = jnp.sum(acc1, axis=1, keepdims=True)
    inf_ref[...] = jnp.concatenate([s0, s1], axis=1)

    lbl = label_ref[...]
    contrib = jnp.where(s1 > s0, lbl[:, 1:2], lbl[:, 0:1])
    partial = jnp.sum(contrib)

    @pl.when(pl.program_id(0) == 0)
    def _():
        rew_ref[0, 0] = 0.0
    rew_ref[0, 0] += partial


def kernel(feat_idx, label, emb_mean, emb_std, fc_w, fc_w_concat,
           mlp_p_W1, mlp_p_b1, mlp_p_W2, mlp_p_b2,
           mlp_q_W1, mlp_q_b1, mlp_q_W2, mlp_q_b2,
           log_alpha, rand_array):
    f32 = jnp.float32
    idx_t = feat_idx.astype(jnp.int32).T
    mean2 = emb_mean.reshape(_NCOL * _NUM_EMB, _D)
    std2 = emb_std.reshape(_NCOL * _NUM_EMB, _D)
    def rbf(x):
        u = jax.lax.bitcast_convert_type(x, jnp.uint32)
        r = (u + jnp.uint32(0x7FFF) + ((u >> 16) & jnp.uint32(1))) \
            & jnp.uint32(0xFFFF0000)
        return jax.lax.bitcast_convert_type(r, f32)

    fcw = rbf(fc_w.reshape(28 * 4 * 2, _D))
    fcc = rbf(fc_w_concat.reshape(28 * 2, 2 * _D))
    mparams = jnp.stack([
        mlp_p_W1[:, 0], mlp_p_b1, rbf(mlp_p_W2[0, :]), jnp.broadcast_to(mlp_p_b2, (8,)),
        mlp_q_W1[:, 0], mlp_q_b1, rbf(mlp_q_W2[0, :]), jnp.broadcast_to(mlp_q_b2, (8,)),
    ]).astype(f32)
    v = rand_array[:_B * _D].reshape(_B, _D)

    grid = (_B // _BT,)
    inf, rew = pl.pallas_call(
        _nasp_kernel,
        grid=grid,
        in_specs=[
            pl.BlockSpec((_BT, _NCOL), lambda i: (i, 0)),
            pl.BlockSpec((_BT, 2), lambda i: (i, 0)),
            pl.BlockSpec((_NCOL * _NUM_EMB, _D), lambda i: (0, 0)),
            pl.BlockSpec((_NCOL * _NUM_EMB, _D), lambda i: (0, 0)),
            pl.BlockSpec((224, _D), lambda i: (0, 0)),
            pl.BlockSpec((56, 2 * _D), lambda i: (0, 0)),
            pl.BlockSpec(memory_space=pltpu.SMEM),
            pl.BlockSpec(memory_space=pltpu.SMEM),
            pl.BlockSpec((_BT, _D), lambda i: (i, 0)),
        ],
        out_specs=[
            pl.BlockSpec((_BT, 2), lambda i: (i, 0)),
            pl.BlockSpec(memory_space=pltpu.SMEM),
        ],
        out_shape=[
            jax.ShapeDtypeStruct((_B, 2), f32),
            jax.ShapeDtypeStruct((1, 1), f32),
        ],
    )(idx_t, label, mean2, std2, fcw, fcc, mparams, log_alpha, v)
    return inf, rew.reshape(())


# speedup vs baseline: 1.7117x; 1.1926x over previous
import jax
import jax.numpy as jnp
from jax.experimental import pallas as pl
from jax.experimental.pallas import tpu as pltpu

_B = 16384
_D = 64
_NCOL = 8
_NUM_EMB = 12
_PAIRS = [(i1, i2) for i1 in range(_NCOL) for i2 in range(i1 + 1, _NCOL)]
_BT = 1024


def _nasp_kernel(idx_ref, label_ref, mean_ref, std_ref, fcw_ref, fcc_ref,
                 mparams_ref, logalpha_ref, v_ref, inf_ref, rew_ref):
    f32 = jnp.float32
    idx = idx_ref[...]
    v = v_ref[...]
    mean_tab = mean_ref[...]
    sp_tab = jnp.log1p(jnp.exp(std_ref[...]))
    fcw = fcw_ref[...]
    fcc = fcc_ref[...]

    def gather12(tab, idx_col):
        out = jnp.broadcast_to(tab[0:1, :], idx_col.shape[:1] + (_D,))
        for r in range(1, _NUM_EMB):
            out = jnp.where(idx_col == r, tab[r:r + 1, :], out)
        return out

    def rb(x):
        return x.astype(jnp.bfloat16).astype(f32)

    def mlp(x, base):
        acc = None
        for h in range(8):
            z = mparams_ref[base, h] * x + mparams_ref[base + 1, h]
            t = jnp.tanh(z)
            term = mparams_ref[base + 2, h] * rb(t)
            acc = term if acc is None else acc + term
        return acc + mparams_ref[base + 3, 0]

    ep = [None] * _NCOL
    eq = [None] * _NCOL
    for c in range(_NCOL):
        idx_c = idx[:, c:c + 1]
        mu = gather12(mean_tab[c * 12:(c + 1) * 12, :], idx_c)
        s = gather12(sp_tab[c * 12:(c + 1) * 12, :], idx_c)
        e = mu + s * v * 0.01
        if c < _NCOL - 1:
            ep[c] = mlp(e, 0)
        if c > 0:
            eq[c] = mlp(e, 4)

    pos = jnp.int32(0)
    best = logalpha_ref[0, 0]
    for j in range(1, 5):
        lj = logalpha_ref[0, j]
        take = lj > best
        pos = jnp.where(take, jnp.int32(j), pos)
        best = jnp.where(take, lj, best)

    def elem_branch(k):
        def branch():
            acc0 = None
            acc1 = None
            for p, (i1, i2) in enumerate(_PAIRS):
                a, b = ep[i1], eq[i2]
                if k == 0:
                    comb = a + b
                elif k == 1:
                    comb = a * b
                elif k == 2:
                    comb = jnp.maximum(a, b)
                else:
                    comb = jnp.minimum(a, b)
                r = p * 8 + k * 2
                cb = rb(comb)
                t0 = cb * fcw[r:r + 1, :]
                t1 = cb * fcw[r + 1:r + 2, :]
                acc0 = t0 if acc0 is None else acc0 + t0
                acc1 = t1 if acc1 is None else acc1 + t1
            return acc0, acc1
        return branch

    def concat_branch():
        epb = [rb(e) if e is not None else None for e in ep]
        eqb = [rb(e) if e is not None else None for e in eq]
        acc0 = None
        acc1 = None
        for p, (i1, i2) in enumerate(_PAIRS):
            a, b = epb[i1], eqb[i2]
            r = p * 2
            t0 = a * fcc[r:r + 1, 0:64] + b * fcc[r:r + 1, 64:128]
            t1 = a * fcc[r + 1:r + 2, 0:64] + b * fcc[r + 1:r + 2, 64:128]
            acc0 = t0 if acc0 is None else acc0 + t0
            acc1 = t1 if acc1 is None else acc1 + t1
        return acc0, acc1

    acc0, acc1 = jax.lax.switch(
        pos, [elem_branch(0), elem_branch(1), elem_branch(2), elem_branch(3),
              concat_branch])
    s0 = jnp.sum(acc0, axis=1, keepdims=True)
    s1 = jnp.sum(acc1, axis=1, keepdims=True)
    inf_ref[...] = jnp.concatenate([s0, s1], axis=1)

    lbl = label_ref[...]
    contrib = jnp.where(s1 > s0, lbl[:, 1:2], lbl[:, 0:1])
    partial = jnp.sum(contrib)

    @pl.when(pl.program_id(0) == 0)
    def _():
        rew_ref[0, 0] = 0.0
    rew_ref[0, 0] += partial


def kernel(feat_idx, label, emb_mean, emb_std, fc_w, fc_w_concat,
           mlp_p_W1, mlp_p_b1, mlp_p_W2, mlp_p_b2,
           mlp_q_W1, mlp_q_b1, mlp_q_W2, mlp_q_b2,
           log_alpha, rand_array):
    f32 = jnp.float32
    idx_t = feat_idx.astype(jnp.int32).T
    mean2 = emb_mean.reshape(_NCOL * _NUM_EMB, _D)
    std2 = emb_std.reshape(_NCOL * _NUM_EMB, _D)
    def rbf(x):
        u = jax.lax.bitcast_convert_type(x, jnp.uint32)
        r = (u + jnp.uint32(0x7FFF) + ((u >> 16) & jnp.uint32(1))) \
            & jnp.uint32(0xFFFF0000)
        return jax.lax.bitcast_convert_type(r, f32)

    fcw = rbf(fc_w.reshape(28 * 4 * 2, _D))
    fcc = rbf(fc_w_concat.reshape(28 * 2, 2 * _D))
    mparams = jnp.stack([
        mlp_p_W1[:, 0], mlp_p_b1, rbf(mlp_p_W2[0, :]), jnp.broadcast_to(mlp_p_b2, (8,)),
        mlp_q_W1[:, 0], mlp_q_b1, rbf(mlp_q_W2[0, :]), jnp.broadcast_to(mlp_q_b2, (8,)),
    ]).astype(f32)
    v = rand_array[:_B * _D].reshape(_B, _D)

    grid = (_B // _BT,)
    inf, rew = pl.pallas_call(
        _nasp_kernel,
        grid=grid,
        in_specs=[
            pl.BlockSpec((_BT, _NCOL), lambda i: (i, 0)),
            pl.BlockSpec((_BT, 2), lambda i: (i, 0)),
            pl.BlockSpec((_NCOL * _NUM_EMB, _D), lambda i: (0, 0)),
            pl.BlockSpec((_NCOL * _NUM_EMB, _D), lambda i: (0, 0)),
            pl.BlockSpec((224, _D), lambda i: (0, 0)),
            pl.BlockSpec((56, 2 * _D), lambda i: (0, 0)),
            pl.BlockSpec(memory_space=pltpu.SMEM),
            pl.BlockSpec(memory_space=pltpu.SMEM),
            pl.BlockSpec((_BT, _D), lambda i: (i, 0)),
        ],
        out_specs=[
            pl.BlockSpec((_BT, 2), lambda i: (i, 0)),
            pl.BlockSpec(memory_space=pltpu.SMEM),
        ],
        out_shape=[
            jax.ShapeDtypeStruct((_B, 2), f32),
            jax.ShapeDtypeStruct((1, 1), f32),
        ],
    )(idx_t, label, mean2, std2, fcw, fcc, mparams, log_alpha, v)
    return inf, rew.reshape(())


# speedup vs baseline: 2.6115x; 1.5257x over previous
import jax
import jax.numpy as jnp
from jax.experimental import pallas as pl
from jax.experimental.pallas import tpu as pltpu

_B = 16384
_D = 64
_NCOL = 8
_NUM_EMB = 12
_PAIRS = [(i1, i2) for i1 in range(_NCOL) for i2 in range(i1 + 1, _NCOL)]
_BT = 1024


def _nasp_kernel(idx_ref, label_ref, mean_ref, std_ref, fcw_ref, fcc_ref,
                 mparams_ref, logalpha_ref, v_ref, inf_ref, rew_ref):
    f32 = jnp.float32
    idx = idx_ref[...]
    v = v_ref[...]
    mean_tab = mean_ref[...]
    sp_tab = jnp.log1p(jnp.exp(std_ref[...]))
    fcw = fcw_ref[...]
    fcc = fcc_ref[...]

    def gather12(tab, idx_col):
        out = jnp.broadcast_to(tab[0:1, :], idx_col.shape[:1] + (_D,))
        for r in range(1, _NUM_EMB):
            out = jnp.where(idx_col == r, tab[r:r + 1, :], out)
        return out

    def rb(x):
        return x.astype(jnp.bfloat16).astype(f32)

    def mlp(x, base):
        acc = None
        for h in range(8):
            z = mparams_ref[base, h] * x + mparams_ref[base + 1, h]
            t = jnp.tanh(z)
            term = mparams_ref[base + 2, h] * rb(t)
            acc = term if acc is None else acc + term
        return acc + mparams_ref[base + 3, 0]

    ep = [None] * _NCOL
    eq = [None] * _NCOL
    for c in range(_NCOL):
        idx_c = idx[:, c:c + 1]
        mu = gather12(mean_tab[c * 12:(c + 1) * 12, :], idx_c)
        s = gather12(sp_tab[c * 12:(c + 1) * 12, :], idx_c)
        e = mu + s * v * 0.01
        if c < _NCOL - 1:
            ep[c] = mlp(e, 0)
        if c > 0:
            eq[c] = mlp(e, 4)

    pos = jnp.int32(0)
    best = logalpha_ref[0, 0]
    for j in range(1, 5):
        lj = logalpha_ref[0, j]
        take = lj > best
        pos = jnp.where(take, jnp.int32(j), pos)
        best = jnp.where(take, lj, best)

    dn = (((1,), (1,)), ((), ()))

    def elem_branch(k):
        def branch():
            acc = None
            for p, (i1, i2) in enumerate(_PAIRS):
                a, b = ep[i1], eq[i2]
                if k == 0:
                    comb = a + b
                elif k == 1:
                    comb = a * b
                elif k == 2:
                    comb = jnp.maximum(a, b)
                else:
                    comb = jnp.minimum(a, b)
                r = p * 8 + k * 2
                d = jax.lax.dot_general(comb.astype(jnp.bfloat16),
                                        fcw[r:r + 2, :], dn,
                                        preferred_element_type=f32)
                acc = d if acc is None else acc + d
            return acc
        return branch

    def concat_branch():
        epb = [e.astype(jnp.bfloat16) if e is not None else None for e in ep]
        eqb = [e.astype(jnp.bfloat16) if e is not None else None for e in eq]
        acc = None
        for p, (i1, i2) in enumerate(_PAIRS):
            a, b = epb[i1], eqb[i2]
            r = p * 2
            d = (jax.lax.dot_general(a, fcc[r:r + 2, 0:64], dn,
                                     preferred_element_type=f32)
                 + jax.lax.dot_general(b, fcc[r:r + 2, 64:128], dn,
                                       preferred_element_type=f32))
            acc = d if acc is None else acc + d
        return acc

    inf = jax.lax.switch(
        pos, [elem_branch(0), elem_branch(1), elem_branch(2), elem_branch(3),
              concat_branch])
    s0 = inf[:, 0:1]
    s1 = inf[:, 1:2]
    inf_ref[...] = inf

    lbl = label_ref[...]
    contrib = jnp.where(s1 > s0, lbl[:, 1:2], lbl[:, 0:1])
    partial = jnp.sum(contrib)

    @pl.when(pl.program_id(0) == 0)
    def _():
        rew_ref[0, 0] = 0.0
    rew_ref[0, 0] += partial


def kernel(feat_idx, label, emb_mean, emb_std, fc_w, fc_w_concat,
           mlp_p_W1, mlp_p_b1, mlp_p_W2, mlp_p_b2,
           mlp_q_W1, mlp_q_b1, mlp_q_W2, mlp_q_b2,
           log_alpha, rand_array):
    f32 = jnp.float32
    idx_t = feat_idx.astype(jnp.int32).T
    mean2 = emb_mean.reshape(_NCOL * _NUM_EMB, _D)
    std2 = emb_std.reshape(_NCOL * _NUM_EMB, _D)
    def rbf(x):
        u = jax.lax.bitcast_convert_type(x, jnp.uint32)
        r = (u + jnp.uint32(0x7FFF) + ((u >> 16) & jnp.uint32(1))) \
            & jnp.uint32(0xFFFF0000)
        return jax.lax.bitcast_convert_type(r, f32)

    fcw = fc_w.reshape(28 * 4 * 2, _D).astype(jnp.bfloat16)
    fcc = jnp.zeros((64, 2 * _D), jnp.bfloat16).at[:56].set(
        fc_w_concat.reshape(28 * 2, 2 * _D).astype(jnp.bfloat16))
    mparams = jnp.stack([
        mlp_p_W1[:, 0], mlp_p_b1, rbf(mlp_p_W2[0, :]), jnp.broadcast_to(mlp_p_b2, (8,)),
        mlp_q_W1[:, 0], mlp_q_b1, rbf(mlp_q_W2[0, :]), jnp.broadcast_to(mlp_q_b2, (8,)),
    ]).astype(f32)
    v = rand_array[:_B * _D].reshape(_B, _D)

    grid = (_B // _BT,)
    inf, rew = pl.pallas_call(
        _nasp_kernel,
        grid=grid,
        in_specs=[
            pl.BlockSpec((_BT, _NCOL), lambda i: (i, 0)),
            pl.BlockSpec((_BT, 2), lambda i: (i, 0)),
            pl.BlockSpec((_NCOL * _NUM_EMB, _D), lambda i: (0, 0)),
            pl.BlockSpec((_NCOL * _NUM_EMB, _D), lambda i: (0, 0)),
            pl.BlockSpec((224, _D), lambda i: (0, 0)),
            pl.BlockSpec((64, 2 * _D), lambda i: (0, 0)),
            pl.BlockSpec(memory_space=pltpu.SMEM),
            pl.BlockSpec(memory_space=pltpu.SMEM),
            pl.BlockSpec((_BT, _D), lambda i: (i, 0)),
        ],
        out_specs=[
            pl.BlockSpec((_BT, 2), lambda i: (i, 0)),
            pl.BlockSpec(memory_space=pltpu.SMEM),
        ],
        out_shape=[
            jax.ShapeDtypeStruct((_B, 2), f32),
            jax.ShapeDtypeStruct((1, 1), f32),
        ],
    )(idx_t, label, mean2, std2, fcw, fcc, mparams, log_alpha, v)
    return inf, rew.reshape(())


# speedup vs baseline: 2.7304x; 1.0455x over previous
import jax
import jax.numpy as jnp
from jax.experimental import pallas as pl
from jax.experimental.pallas import tpu as pltpu

_B = 16384
_D = 64
_NCOL = 8
_NUM_EMB = 12
_PAIRS = [(i1, i2) for i1 in range(_NCOL) for i2 in range(i1 + 1, _NCOL)]
_BT = 2048


def _nasp_kernel(idx_ref, label_ref, mean_ref, std_ref, fcw_ref, fcc_ref,
                 mparams_ref, logalpha_ref, v_ref, inf_ref, rew_ref):
    f32 = jnp.float32
    idx = idx_ref[...]
    v = v_ref[...]
    mean_tab = mean_ref[...]
    sp_tab = jnp.log1p(jnp.exp(std_ref[...]))
    fcw = fcw_ref[...]
    fcc = fcc_ref[...]

    def gather12(tab, idx_col):
        out = jnp.broadcast_to(tab[0:1, :], idx_col.shape[:1] + (_D,))
        for r in range(1, _NUM_EMB):
            out = jnp.where(idx_col == r, tab[r:r + 1, :], out)
        return out

    def rb(x):
        return x.astype(jnp.bfloat16).astype(f32)

    def mlp(x, base):
        acc = None
        for h in range(8):
            z = mparams_ref[base, h] * x + mparams_ref[base + 1, h]
            t = jnp.tanh(z)
            term = mparams_ref[base + 2, h] * rb(t)
            acc = term if acc is None else acc + term
        return acc + mparams_ref[base + 3, 0]

    ep = [None] * _NCOL
    eq = [None] * _NCOL
    for c in range(_NCOL):
        idx_c = idx[:, c:c + 1]
        mu = gather12(mean_tab[c * 12:(c + 1) * 12, :], idx_c)
        s = gather12(sp_tab[c * 12:(c + 1) * 12, :], idx_c)
        e = mu + s * v * 0.01
        if c < _NCOL - 1:
            ep[c] = mlp(e, 0)
        if c > 0:
            eq[c] = mlp(e, 4)

    pos = jnp.int32(0)
    best = logalpha_ref[0, 0]
    for j in range(1, 5):
        lj = logalpha_ref[0, j]
        take = lj > best
        pos = jnp.where(take, jnp.int32(j), pos)
        best = jnp.where(take, lj, best)

    dn = (((1,), (1,)), ((), ()))

    def elem_branch(k):
        def branch():
            acc = None
            for p, (i1, i2) in enumerate(_PAIRS):
                a, b = ep[i1], eq[i2]
                if k == 0:
                    comb = a + b
                elif k == 1:
                    comb = a * b
                elif k == 2:
                    comb = jnp.maximum(a, b)
                else:
                    comb = jnp.minimum(a, b)
                r = p * 8 + k * 2
                d = jax.lax.dot_general(comb.astype(jnp.bfloat16),
                                        fcw[r:r + 2, :], dn,
                                        preferred_element_type=f32)
                acc = d if acc is None else acc + d
            return acc
        return branch

    def concat_branch():
        epb = [e.astype(jnp.bfloat16) if e is not None else None for e in ep]
        eqb = [e.astype(jnp.bfloat16) if e is not None else None for e in eq]
        acc = None
        for p, (i1, i2) in enumerate(_PAIRS):
            a, b = epb[i1], eqb[i2]
            r = p * 2
            d = (jax.lax.dot_general(a, fcc[r:r + 2, 0:64], dn,
                                     preferred_element_type=f32)
                 + jax.lax.dot_general(b, fcc[r:r + 2, 64:128], dn,
                                       preferred_element_type=f32))
            acc = d if acc is None else acc + d
        return acc

    inf = jax.lax.switch(
        pos, [elem_branch(0), elem_branch(1), elem_branch(2), elem_branch(3),
              concat_branch])
    s0 = inf[:, 0:1]
    s1 = inf[:, 1:2]
    inf_ref[...] = inf

    lbl = label_ref[...]
    contrib = jnp.where(s1 > s0, lbl[:, 1:2], lbl[:, 0:1])
    partial = jnp.sum(contrib)

    @pl.when(pl.program_id(0) == 0)
    def _():
        rew_ref[0, 0] = 0.0
    rew_ref[0, 0] += partial


def kernel(feat_idx, label, emb_mean, emb_std, fc_w, fc_w_concat,
           mlp_p_W1, mlp_p_b1, mlp_p_W2, mlp_p_b2,
           mlp_q_W1, mlp_q_b1, mlp_q_W2, mlp_q_b2,
           log_alpha, rand_array):
    f32 = jnp.float32
    idx_t = feat_idx.astype(jnp.int32).T
    mean2 = emb_mean.reshape(_NCOL * _NUM_EMB, _D)
    std2 = emb_std.reshape(_NCOL * _NUM_EMB, _D)
    def rbf(x):
        u = jax.lax.bitcast_convert_type(x, jnp.uint32)
        r = (u + jnp.uint32(0x7FFF) + ((u >> 16) & jnp.uint32(1))) \
            & jnp.uint32(0xFFFF0000)
        return jax.lax.bitcast_convert_type(r, f32)

    fcw = fc_w.reshape(28 * 4 * 2, _D).astype(jnp.bfloat16)
    fcc = jnp.zeros((64, 2 * _D), jnp.bfloat16).at[:56].set(
        fc_w_concat.reshape(28 * 2, 2 * _D).astype(jnp.bfloat16))
    mparams = jnp.stack([
        mlp_p_W1[:, 0], mlp_p_b1, rbf(mlp_p_W2[0, :]), jnp.broadcast_to(mlp_p_b2, (8,)),
        mlp_q_W1[:, 0], mlp_q_b1, rbf(mlp_q_W2[0, :]), jnp.broadcast_to(mlp_q_b2, (8,)),
    ]).astype(f32)
    v = rand_array[:_B * _D].reshape(_B, _D)

    grid = (_B // _BT,)
    inf, rew = pl.pallas_call(
        _nasp_kernel,
        grid=grid,
        in_specs=[
            pl.BlockSpec((_BT, _NCOL), lambda i: (i, 0)),
            pl.BlockSpec((_BT, 2), lambda i: (i, 0)),
            pl.BlockSpec((_NCOL * _NUM_EMB, _D), lambda i: (0, 0)),
            pl.BlockSpec((_NCOL * _NUM_EMB, _D), lambda i: (0, 0)),
            pl.BlockSpec((224, _D), lambda i: (0, 0)),
            pl.BlockSpec((64, 2 * _D), lambda i: (0, 0)),
            pl.BlockSpec(memory_space=pltpu.SMEM),
            pl.BlockSpec(memory_space=pltpu.SMEM),
            pl.BlockSpec((_BT, _D), lambda i: (i, 0)),
        ],
        out_specs=[
            pl.BlockSpec((_BT, 2), lambda i: (i, 0)),
            pl.BlockSpec(memory_space=pltpu.SMEM),
        ],
        out_shape=[
            jax.ShapeDtypeStruct((_B, 2), f32),
            jax.ShapeDtypeStruct((1, 1), f32),
        ],
    )(idx_t, label, mean2, std2, fcw, fcc, mparams, log_alpha, v)
    return inf, rew.reshape(())


# speedup vs baseline: 3.6463x; 1.3355x over previous
import jax
import jax.numpy as jnp
from jax.experimental import pallas as pl
from jax.experimental.pallas import tpu as pltpu

_B = 16384
_D = 64
_NCOL = 8
_NUM_EMB = 12
_PAIRS = [(i1, i2) for i1 in range(_NCOL) for i2 in range(i1 + 1, _NCOL)]
_BT = 2048


def _nasp_kernel(idx_ref, label_ref, mean_ref, std_ref, fcw_ref, fcc_ref,
                 mparams_ref, logalpha_ref, v_ref, inf_ref, rew_ref):
    f32 = jnp.float32
    idx = idx_ref[...]
    v = v_ref[...]
    mean_tab = mean_ref[...]
    sp_tab = jnp.log1p(jnp.exp(std_ref[...]))
    fcw = fcw_ref[...]
    fcc = fcc_ref[...]

    bf16 = jnp.bfloat16
    mean_hi = mean_tab.astype(bf16)
    mean_r1 = mean_tab - mean_hi.astype(f32)
    mean_mid = mean_r1.astype(bf16)
    mean_lo = (mean_r1 - mean_mid.astype(f32)).astype(bf16)
    sp_hi = sp_tab.astype(bf16)
    sp_lo = (sp_tab - sp_hi.astype(f32)).astype(bf16)
    iota12 = jax.lax.broadcasted_iota(jnp.int32, (_BT, _NUM_EMB), 1)

    def odot(oh, tab):
        return jnp.dot(oh, tab, preferred_element_type=f32)

    def rb(x):
        return x.astype(jnp.bfloat16).astype(f32)

    def mlp(x, base):
        acc = None
        for h in range(8):
            z = mparams_ref[base, h] * x + mparams_ref[base + 1, h]
            t = jnp.tanh(z)
            term = mparams_ref[base + 2, h] * rb(t)
            acc = term if acc is None else acc + term
        return acc + mparams_ref[base + 3, 0]

    ep = [None] * _NCOL
    eq = [None] * _NCOL
    for c in range(_NCOL):
        oh = (idx[:, c:c + 1] == iota12).astype(jnp.bfloat16)
        sl = slice(c * 12, (c + 1) * 12)
        mu = (odot(oh, mean_hi[sl]) + odot(oh, mean_mid[sl])
              + odot(oh, mean_lo[sl]))
        s = odot(oh, sp_hi[sl]) + odot(oh, sp_lo[sl])
        e = mu + s * v * 0.01
        if c < _NCOL - 1:
            ep[c] = mlp(e, 0)
        if c > 0:
            eq[c] = mlp(e, 4)

    pos = jnp.int32(0)
    best = logalpha_ref[0, 0]
    for j in range(1, 5):
        lj = logalpha_ref[0, j]
        take = lj > best
        pos = jnp.where(take, jnp.int32(j), pos)
        best = jnp.where(take, lj, best)

    dn = (((1,), (1,)), ((), ()))

    def elem_branch(k):
        def branch():
            acc = None
            for p, (i1, i2) in enumerate(_PAIRS):
                a, b = ep[i1], eq[i2]
                if k == 0:
                    comb = a + b
                elif k == 1:
                    comb = a * b
                elif k == 2:
                    comb = jnp.maximum(a, b)
                else:
                    comb = jnp.minimum(a, b)
                r = p * 8 + k * 2
                d = jax.lax.dot_general(comb.astype(jnp.bfloat16),
                                        fcw[r:r + 2, :], dn,
                                        preferred_element_type=f32)
                acc = d if acc is None else acc + d
            return acc
        return branch

    def concat_branch():
        epb = [e.astype(jnp.bfloat16) if e is not None else None for e in ep]
        eqb = [e.astype(jnp.bfloat16) if e is not None else None for e in eq]
        acc = None
        for p, (i1, i2) in enumerate(_PAIRS):
            a, b = epb[i1], eqb[i2]
            r = p * 2
            d = (jax.lax.dot_general(a, fcc[r:r + 2, 0:64], dn,
                                     preferred_element_type=f32)
                 + jax.lax.dot_general(b, fcc[r:r + 2, 64:128], dn,
                                       preferred_element_type=f32))
            acc = d if acc is None else acc + d
        return acc

    inf = jax.lax.switch(
        pos, [elem_branch(0), elem_branch(1), elem_branch(2), elem_branch(3),
              concat_branch])
    s0 = inf[:, 0:1]
    s1 = inf[:, 1:2]
    inf_ref[...] = inf

    lbl = label_ref[...]
    contrib = jnp.where(s1 > s0, lbl[:, 1:2], lbl[:, 0:1])
    partial = jnp.sum(contrib)

    @pl.when(pl.program_id(0) == 0)
    def _():
        rew_ref[0, 0] = 0.0
    rew_ref[0, 0] += partial


def kernel(feat_idx, label, emb_mean, emb_std, fc_w, fc_w_concat,
           mlp_p_W1, mlp_p_b1, mlp_p_W2, mlp_p_b2,
           mlp_q_W1, mlp_q_b1, mlp_q_W2, mlp_q_b2,
           log_alpha, rand_array):
    f32 = jnp.float32
    idx_t = feat_idx.astype(jnp.int32).T
    mean2 = emb_mean.reshape(_NCOL * _NUM_EMB, _D)
    std2 = emb_std.reshape(_NCOL * _NUM_EMB, _D)
    def rbf(x):
        u = jax.lax.bitcast_convert_type(x, jnp.uint32)
        r = (u + jnp.uint32(0x7FFF) + ((u >> 16) & jnp.uint32(1))) \
            & jnp.uint32(0xFFFF0000)
        return jax.lax.bitcast_convert_type(r, f32)

    fcw = fc_w.reshape(28 * 4 * 2, _D).astype(jnp.bfloat16)
    fcc = jnp.zeros((64, 2 * _D), jnp.bfloat16).at[:56].set(
        fc_w_concat.reshape(28 * 2, 2 * _D).astype(jnp.bfloat16))
    mparams = jnp.stack([
        mlp_p_W1[:, 0], mlp_p_b1, rbf(mlp_p_W2[0, :]), jnp.broadcast_to(mlp_p_b2, (8,)),
        mlp_q_W1[:, 0], mlp_q_b1, rbf(mlp_q_W2[0, :]), jnp.broadcast_to(mlp_q_b2, (8,)),
    ]).astype(f32)
    v = rand_array[:_B * _D].reshape(_B, _D)

    grid = (_B // _BT,)
    inf, rew = pl.pallas_call(
        _nasp_kernel,
        grid=grid,
        in_specs=[
            pl.BlockSpec((_BT, _NCOL), lambda i: (i, 0)),
            pl.BlockSpec((_BT, 2), lambda i: (i, 0)),
            pl.BlockSpec((_NCOL * _NUM_EMB, _D), lambda i: (0, 0)),
            pl.BlockSpec((_NCOL * _NUM_EMB, _D), lambda i: (0, 0)),
            pl.BlockSpec((224, _D), lambda i: (0, 0)),
            pl.BlockSpec((64, 2 * _D), lambda i: (0, 0)),
            pl.BlockSpec(memory_space=pltpu.SMEM),
            pl.BlockSpec(memory_space=pltpu.SMEM),
            pl.BlockSpec((_BT, _D), lambda i: (i, 0)),
        ],
        out_specs=[
            pl.BlockSpec((_BT, 2), lambda i: (i, 0)),
            pl.BlockSpec(memory_space=pltpu.SMEM),
        ],
        out_shape=[
            jax.ShapeDtypeStruct((_B, 2), f32),
            jax.ShapeDtypeStruct((1, 1), f32),
        ],
    )(idx_t, label, mean2, std2, fcw, fcc, mparams, log_alpha, v)
    return inf, rew.reshape(())
